# baseline (speedup 1.0000x reference)
import functools

import jax
import jax.numpy as jnp
from jax.experimental import pallas as pl
from jax.experimental.pallas import tpu as pltpu

_K_TOP = 128
_LOCAL_K = 64
_G = 4
_NEG = -3.0e38


def _ann_body(qp_ref, w_ref, k_ref, v_ref, kg_ref, vg_ref, mv_ref, rem_ref,
              *, S, D, local_k, n_top, G):
    n_nl = n_top - local_k
    f32 = jnp.float32
    bf = jnp.bfloat16
    rows = S // 128
    big = jnp.int32(2147483647)

    flat = (jax.lax.broadcasted_iota(jnp.int32, (rows, 128), 0) * 128
            + jax.lax.broadcasted_iota(jnp.int32, (rows, 128), 1))
    is_local = flat >= (S - local_k)
    sub_j = jax.lax.broadcasted_iota(jnp.int32, (n_nl, 1), 0)
    oh_lane = jax.lax.broadcasted_iota(jnp.int32, (n_nl, S), 1)

    kb, vb, ms0, e_all, se_all, m_all = [], [], [], [], [], []
    for g in range(G):
        Wb = w_ref[g].astype(bf)
        qpg = qp_ref[g].astype(bf)
        k2 = k_ref[g]
        v2 = v_ref[g]
        kbg = k2.astype(bf)
        vbg = v2.astype(bf)
        kp = jax.lax.dot_general(kbg, Wb, (((1,), (0,)), ((), ())),
                                 preferred_element_type=f32)
        sc = jax.lax.dot_general(kp.astype(bf), qpg, (((1,), (1,)), ((), ())),
                                 preferred_element_type=f32)[:, 0:1]
        sc = (sc * (D ** -0.5)).reshape(rows, 128)
        m = jnp.max(jnp.max(sc, axis=0, keepdims=True), axis=1,
                    keepdims=True)
        e = jnp.exp(sc - m)
        se = jnp.sum(jnp.sum(e, axis=0, keepdims=True), axis=1,
                     keepdims=True)
        kb.append(kbg)
        vb.append(vbg)
        ms0.append(jnp.where(is_local, _NEG, sc))
        e_all.append(e)
        se_all.append(se)
        m_all.append(m)

        kg_ref[g, 0:local_k, :] = k_ref[g, S - local_k:S, :]
        vg_ref[g, 0:local_k, :] = v_ref[g, S - local_k:S, :]
        mv_ref[g, :, :] = jnp.sum(v2, axis=0, keepdims=True) * (1.0 / S)

    def body(j, carry):
        ms, acc, idxc = carry
        ms_n, acc_n, idxc_n = [], [], []
        for g in range(G):
            mj = jnp.max(jnp.max(ms[g], axis=0, keepdims=True), axis=1,
                         keepdims=True)
            cand = jnp.where(ms[g] == mj, flat, big)
            i11 = jnp.min(jnp.min(cand, axis=0, keepdims=True), axis=1,
                          keepdims=True)
            idxc_n.append(jnp.where(sub_j == j, i11, idxc[g]))
            acc_n.append(acc[g] + jnp.exp(mj - m_all[g]))
            ms_n.append(jnp.where(flat == i11, _NEG, ms[g]))
        return tuple(ms_n), tuple(acc_n), tuple(idxc_n)

    init = (tuple(ms0),
            tuple(jnp.zeros((1, 1), f32) for _ in range(G)),
            tuple(jnp.zeros((n_nl, 1), jnp.int32) for _ in range(G)))
    _, accs, idxcs = jax.lax.fori_loop(0, n_nl, body, init)

    for g in range(G):
        oh = jnp.where(oh_lane == idxcs[g], 1.0, 0.0).astype(bf)
        rows_k = jax.lax.dot_general(oh, kb[g], (((1,), (0,)), ((), ())),
                                     preferred_element_type=f32)
        rows_v = jax.lax.dot_general(oh, vb[g], (((1,), (0,)), ((), ())),
                                     preferred_element_type=f32)
        kg_ref[g, local_k:n_top, :] = rows_k
        vg_ref[g, local_k:n_top, :] = rows_v

        p_loc = (jnp.sum(jnp.sum(jnp.where(is_local, e_all[g], 0.0), axis=0,
                                 keepdims=True), axis=1, keepdims=True)
                 / se_all[g])
        p_nl = accs[g] / se_all[g]
        norm = m_all[g] + jnp.log(se_all[g])
        remainder = jnp.log(1.0 - (p_loc + p_nl)) + norm
        rem_ref[g, :, :] = jnp.broadcast_to(remainder, (1, 128))


def kernel(query, key, value, logmask, W):
    B, H, _, D = query.shape
    S = key.shape[2]
    BH = B * H
    R = W.shape[-1]
    G = _G
    qp = jnp.matmul(query, W).reshape(BH, 1, R)
    qp = jnp.broadcast_to(qp, (BH, 8, R))
    k = key.reshape(BH, S, D)
    v = value.reshape(BH, S, D)

    body = functools.partial(_ann_body, S=S, D=D,
                             local_k=_LOCAL_K, n_top=_K_TOP, G=G)
    kg, vg, mv, rem = pl.pallas_call(
        body,
        grid=(BH // G,),
        in_specs=[
            pl.BlockSpec((G, 8, R), lambda i: (i, 0, 0)),
            pl.BlockSpec((G, D, R), lambda i: (i % (H // G), 0, 0)),
            pl.BlockSpec((G, S, D), lambda i: (i, 0, 0)),
            pl.BlockSpec((G, S, D), lambda i: (i, 0, 0)),
        ],
        out_specs=[
            pl.BlockSpec((G, _K_TOP, D), lambda i: (i, 0, 0)),
            pl.BlockSpec((G, _K_TOP, D), lambda i: (i, 0, 0)),
            pl.BlockSpec((G, 1, D), lambda i: (i, 0, 0)),
            pl.BlockSpec((G, 1, D), lambda i: (i, 0, 0)),
        ],
        out_shape=[
            jax.ShapeDtypeStruct((BH, _K_TOP, D), jnp.float32),
            jax.ShapeDtypeStruct((BH, _K_TOP, D), jnp.float32),
            jax.ShapeDtypeStruct((BH, 1, D), jnp.float32),
            jax.ShapeDtypeStruct((BH, 1, D), jnp.float32),
        ],
        compiler_params=pltpu.CompilerParams(
            dimension_semantics=("arbitrary",)),
    )(qp, W, k, v)

    zeros_row = jnp.zeros((B, H, 1, D), jnp.float32)
    key_out = jnp.concatenate(
        [zeros_row, kg.reshape(B, H, _K_TOP, D)], axis=-2)
    value_out = jnp.concatenate(
        [mv.reshape(B, H, 1, D), vg.reshape(B, H, _K_TOP, D)], axis=-2)
    logmask_out = jnp.concatenate(
        [rem.reshape(B, H, 1, D)[..., :1],
         jnp.zeros((B, H, 1, _K_TOP), logmask.dtype)], axis=-1)
    return (query, key_out, value_out, logmask_out)


# speedup vs baseline: 1.7589x; 1.7589x over previous
import functools

import jax
import jax.numpy as jnp
from jax.experimental import pallas as pl
from jax.experimental.pallas import tpu as pltpu

_K_TOP = 128
_LOCAL_K = 64
_G = 8
_NEG = -3.0e38


def _score_body(qp_ref, w_ref, k_ref, kg_ref, idx_ref, rem_ref,
                *, S, D, local_k, n_top, G):
    n_nl = n_top - local_k
    f32 = jnp.float32
    bf = jnp.bfloat16
    big = jnp.int32(2147483647)

    lane = jax.lax.broadcasted_iota(jnp.int32, (G, S), 1)
    is_local = lane >= (S - local_k)
    row = jax.lax.broadcasted_iota(jnp.int32, (G, S), 0)
    lane64 = jax.lax.broadcasted_iota(jnp.int32, (G, n_nl), 1)
    oh_lane = jax.lax.broadcasted_iota(jnp.int32, (n_nl, S), 1)

    kb = []
    scores = None
    for g in range(G):
        Wb = w_ref[g].astype(bf)
        kbg = k_ref[g].astype(bf)
        kp = jax.lax.dot_general(kbg, Wb, (((1,), (0,)), ((), ())),
                                 preferred_element_type=f32)
        qpg = qp_ref[g].astype(bf)
        sf = jax.lax.dot_general(qpg, kp.astype(bf), (((1,), (1,)), ((), ())),
                                 preferred_element_type=f32)
        scores = sf if g == 0 else jnp.where(row == g, sf, scores)
        kb.append(kbg)
        kg_ref[g, 0:local_k, :] = k_ref[g, S - local_k:S, :]

    scores = scores * (D ** -0.5)
    m8 = jnp.max(scores, axis=1, keepdims=True)
    e8 = jnp.exp(scores - m8)
    se8 = jnp.sum(e8, axis=1, keepdims=True)
    p_loc = jnp.sum(jnp.where(is_local, e8, 0.0), axis=1,
                    keepdims=True) / se8
    ms0 = jnp.where(is_local, _NEG, scores)

    def body(j, carry):
        ms, acc, idxc = carry
        mj = jnp.max(ms, axis=1, keepdims=True)
        cand = jnp.where(ms == mj, lane, big)
        i8 = jnp.min(cand, axis=1, keepdims=True)
        idxc = jnp.where(lane64 == j, i8, idxc)
        acc = acc + jnp.exp(mj - m8)
        ms = jnp.where(lane == i8, _NEG, ms)
        return ms, acc, idxc

    _, acc8, idxc8 = jax.lax.fori_loop(
        0, n_nl, body,
        (ms0, jnp.zeros((G, 1), f32), jnp.zeros((G, n_nl), jnp.int32)))

    for g in range(G):
        idxcol = idxc8[g:g + 1, :].reshape(n_nl, 1)
        oh = jnp.where(oh_lane == idxcol, 1.0, 0.0).astype(bf)
        rows_k = jax.lax.dot_general(oh, kb[g], (((1,), (0,)), ((), ())),
                                     preferred_element_type=f32)
        kg_ref[g, local_k:n_top, :] = rows_k

    norm = m8 + jnp.log(se8)
    remainder = jnp.log(1.0 - (p_loc + acc8 / se8)) + norm
    rem_ref[0] = jnp.broadcast_to(remainder, (G, 128))
    idx_ref[0] = idxc8


def _value_body(idx_ref, v_ref, vg_ref, mv_ref, *, S, D, local_k, n_top, G):
    n_nl = n_top - local_k
    f32 = jnp.float32
    bf = jnp.bfloat16
    oh_lane = jax.lax.broadcasted_iota(jnp.int32, (n_nl, S), 1)
    for g in range(G):
        v2 = v_ref[g]
        mv_ref[0, g:g + 1, :] = jnp.sum(v2, axis=0, keepdims=True) * (1.0 / S)
        vg_ref[g, 0:local_k, :] = v_ref[g, S - local_k:S, :]
        idxcol = idx_ref[0][g:g + 1, :].reshape(n_nl, 1)
        oh = jnp.where(oh_lane == idxcol, 1.0, 0.0).astype(bf)
        rows_v = jax.lax.dot_general(oh, v2.astype(bf), (((1,), (0,)), ((), ())),
                                     preferred_element_type=f32)
        vg_ref[g, local_k:n_top, :] = rows_v


def kernel(query, key, value, logmask, W):
    B, H, _, D = query.shape
    S = key.shape[2]
    BH = B * H
    R = W.shape[-1]
    G = _G
    NS = BH // G
    qp = jnp.matmul(query, W).reshape(BH, 1, R)
    qp = jnp.broadcast_to(qp, (BH, 8, R))
    k = key.reshape(BH, S, D)
    v = value.reshape(BH, S, D)

    sbody = functools.partial(_score_body, S=S, D=D,
                              local_k=_LOCAL_K, n_top=_K_TOP, G=G)
    kg, idx, rem = pl.pallas_call(
        sbody,
        grid=(NS,),
        in_specs=[
            pl.BlockSpec((G, 8, R), lambda i: (i, 0, 0)),
            pl.BlockSpec((G, D, R), lambda i: (i % (H // G), 0, 0)),
            pl.BlockSpec((G, S, D), lambda i: (i, 0, 0)),
        ],
        out_specs=[
            pl.BlockSpec((G, _K_TOP, D), lambda i: (i, 0, 0)),
            pl.BlockSpec((1, G, _K_TOP - _LOCAL_K), lambda i: (i, 0, 0)),
            pl.BlockSpec((1, G, D), lambda i: (i, 0, 0)),
        ],
        out_shape=[
            jax.ShapeDtypeStruct((BH, _K_TOP, D), jnp.float32),
            jax.ShapeDtypeStruct((NS, G, _K_TOP - _LOCAL_K), jnp.int32),
            jax.ShapeDtypeStruct((NS, G, D), jnp.float32),
        ],
        compiler_params=pltpu.CompilerParams(
            dimension_semantics=("arbitrary",)),
    )(qp, W, k)

    vbody = functools.partial(_value_body, S=S, D=D,
                              local_k=_LOCAL_K, n_top=_K_TOP, G=G)
    vg, mv = pl.pallas_call(
        vbody,
        grid=(NS,),
        in_specs=[
            pl.BlockSpec((1, G, _K_TOP - _LOCAL_K), lambda i: (i, 0, 0)),
            pl.BlockSpec((G, S, D), lambda i: (i, 0, 0)),
        ],
        out_specs=[
            pl.BlockSpec((G, _K_TOP, D), lambda i: (i, 0, 0)),
            pl.BlockSpec((1, G, D), lambda i: (i, 0, 0)),
        ],
        out_shape=[
            jax.ShapeDtypeStruct((BH, _K_TOP, D), jnp.float32),
            jax.ShapeDtypeStruct((NS, G, D), jnp.float32),
        ],
        compiler_params=pltpu.CompilerParams(
            dimension_semantics=("arbitrary",)),
    )(idx, v)

    zeros_row = jnp.zeros((B, H, 1, D), jnp.float32)
    key_out = jnp.concatenate(
        [zeros_row, kg.reshape(B, H, _K_TOP, D)], axis=-2)
    value_out = jnp.concatenate(
        [mv.reshape(B, H, 1, D), vg.reshape(B, H, _K_TOP, D)], axis=-2)
    logmask_out = jnp.concatenate(
        [rem.reshape(B, H, 1, D)[..., :1],
         jnp.zeros((B, H, 1, _K_TOP), logmask.dtype)], axis=-1)
    return (query, key_out, value_out, logmask_out)


# speedup vs baseline: 1.7601x; 1.0007x over previous
import functools

import jax
import jax.numpy as jnp
from jax.experimental import pallas as pl
from jax.experimental.pallas import tpu as pltpu

_K_TOP = 128
_LOCAL_K = 64
_G = 8
_NEG = -3.0e38


def _score_body(qp_ref, w_ref, k_ref, kg_ref, idx_ref, rem_ref,
                *, S, D, local_k, n_top, G):
    n_nl = n_top - local_k
    f32 = jnp.float32
    bf = jnp.bfloat16
    big = jnp.int32(2147483647)

    lane = jax.lax.broadcasted_iota(jnp.int32, (G, S), 1)
    is_local = lane >= (S - local_k)
    row = jax.lax.broadcasted_iota(jnp.int32, (G, S), 0)
    lane64 = jax.lax.broadcasted_iota(jnp.int32, (G, n_nl), 1)
    oh_lane = jax.lax.broadcasted_iota(jnp.int32, (n_nl, S), 1)

    kb = []
    scores = None
    for g in range(G):
        Wb = w_ref[g].astype(bf)
        kbg = k_ref[g].astype(bf)
        kp = jax.lax.dot_general(kbg, Wb, (((1,), (0,)), ((), ())),
                                 preferred_element_type=f32)
        qpg = qp_ref[g].astype(bf)
        sf = jax.lax.dot_general(qpg, kp.astype(bf), (((1,), (1,)), ((), ())),
                                 preferred_element_type=f32)
        scores = sf if g == 0 else jnp.where(row == g, sf, scores)
        kb.append(kbg)
        kg_ref[g, 0:local_k, :] = k_ref[g, S - local_k:S, :]

    scores = scores * (D ** -0.5)
    m8 = jnp.max(scores, axis=1, keepdims=True)
    e8 = jnp.exp(scores - m8)
    se8 = jnp.sum(e8, axis=1, keepdims=True)
    p_loc = jnp.sum(jnp.where(is_local, e8, 0.0), axis=1,
                    keepdims=True) / se8
    ms0 = jnp.where(is_local, _NEG, scores)

    def body(j, carry):
        ms, mjs, idxc = carry
        mj = jnp.max(ms, axis=1, keepdims=True)
        cand = jnp.where(ms == mj, lane, big)
        i8 = jnp.min(cand, axis=1, keepdims=True)
        idxc = jnp.where(lane64 == j, i8, idxc)
        mjs = jnp.where(lane64 == j, mj, mjs)
        ms = jnp.where(lane == i8, _NEG, ms)
        return ms, mjs, idxc

    _, mjs8, idxc8 = jax.lax.fori_loop(
        0, n_nl, body,
        (ms0, jnp.zeros((G, n_nl), f32), jnp.zeros((G, n_nl), jnp.int32)))
    acc8 = jnp.sum(jnp.exp(mjs8 - m8), axis=1, keepdims=True)

    for g in range(G):
        idxcol = idxc8[g:g + 1, :].reshape(n_nl, 1)
        oh = jnp.where(oh_lane == idxcol, 1.0, 0.0).astype(bf)
        rows_k = jax.lax.dot_general(oh, kb[g], (((1,), (0,)), ((), ())),
                                     preferred_element_type=f32)
        kg_ref[g, local_k:n_top, :] = rows_k

    norm = m8 + jnp.log(se8)
    remainder = jnp.log(1.0 - (p_loc + acc8 / se8)) + norm
    rem_ref[0] = jnp.broadcast_to(remainder, (G, 128))
    idx_ref[0] = idxc8


def _value_body(idx_ref, v_ref, vg_ref, mv_ref, *, S, D, local_k, n_top, G):
    n_nl = n_top - local_k
    f32 = jnp.float32
    bf = jnp.bfloat16
    oh_lane = jax.lax.broadcasted_iota(jnp.int32, (n_nl, S), 1)
    for g in range(G):
        v2 = v_ref[g]
        mv_ref[0, g:g + 1, :] = jnp.sum(v2, axis=0, keepdims=True) * (1.0 / S)
        vg_ref[g, 0:local_k, :] = v_ref[g, S - local_k:S, :]
        idxcol = idx_ref[0][g:g + 1, :].reshape(n_nl, 1)
        oh = jnp.where(oh_lane == idxcol, 1.0, 0.0).astype(bf)
        rows_v = jax.lax.dot_general(oh, v2.astype(bf), (((1,), (0,)), ((), ())),
                                     preferred_element_type=f32)
        vg_ref[g, local_k:n_top, :] = rows_v


def kernel(query, key, value, logmask, W):
    B, H, _, D = query.shape
    S = key.shape[2]
    BH = B * H
    R = W.shape[-1]
    G = _G
    NS = BH // G
    qp = jnp.matmul(query, W).reshape(BH, 1, R)
    qp = jnp.broadcast_to(qp, (BH, 8, R))
    k = key.reshape(BH, S, D)
    v = value.reshape(BH, S, D)

    sbody = functools.partial(_score_body, S=S, D=D,
                              local_k=_LOCAL_K, n_top=_K_TOP, G=G)
    kg, idx, rem = pl.pallas_call(
        sbody,
        grid=(NS,),
        in_specs=[
            pl.BlockSpec((G, 8, R), lambda i: (i, 0, 0)),
            pl.BlockSpec((G, D, R), lambda i: (i % (H // G), 0, 0)),
            pl.BlockSpec((G, S, D), lambda i: (i, 0, 0)),
        ],
        out_specs=[
            pl.BlockSpec((G, _K_TOP, D), lambda i: (i, 0, 0)),
            pl.BlockSpec((1, G, _K_TOP - _LOCAL_K), lambda i: (i, 0, 0)),
            pl.BlockSpec((1, G, D), lambda i: (i, 0, 0)),
        ],
        out_shape=[
            jax.ShapeDtypeStruct((BH, _K_TOP, D), jnp.float32),
            jax.ShapeDtypeStruct((NS, G, _K_TOP - _LOCAL_K), jnp.int32),
            jax.ShapeDtypeStruct((NS, G, D), jnp.float32),
        ],
        compiler_params=pltpu.CompilerParams(
            dimension_semantics=("arbitrary",)),
    )(qp, W, k)

    vbody = functools.partial(_value_body, S=S, D=D,
                              local_k=_LOCAL_K, n_top=_K_TOP, G=G)
    vg, mv = pl.pallas_call(
        vbody,
        grid=(NS,),
        in_specs=[
            pl.BlockSpec((1, G, _K_TOP - _LOCAL_K), lambda i: (i, 0, 0)),
            pl.BlockSpec((G, S, D), lambda i: (i, 0, 0)),
        ],
        out_specs=[
            pl.BlockSpec((G, _K_TOP, D), lambda i: (i, 0, 0)),
            pl.BlockSpec((1, G, D), lambda i: (i, 0, 0)),
        ],
        out_shape=[
            jax.ShapeDtypeStruct((BH, _K_TOP, D), jnp.float32),
            jax.ShapeDtypeStruct((NS, G, D), jnp.float32),
        ],
        compiler_params=pltpu.CompilerParams(
            dimension_semantics=("arbitrary",)),
    )(idx, v)

    zeros_row = jnp.zeros((B, H, 1, D), jnp.float32)
    key_out = jnp.concatenate(
        [zeros_row, kg.reshape(B, H, _K_TOP, D)], axis=-2)
    value_out = jnp.concatenate(
        [mv.reshape(B, H, 1, D), vg.reshape(B, H, _K_TOP, D)], axis=-2)
    logmask_out = jnp.concatenate(
        [rem.reshape(B, H, 1, D)[..., :1],
         jnp.zeros((B, H, 1, _K_TOP), logmask.dtype)], axis=-1)
    return (query, key_out, value_out, logmask_out)


# speedup vs baseline: 1.7876x; 1.0156x over previous
import functools

import jax
import jax.numpy as jnp
from jax.experimental import pallas as pl
from jax.experimental.pallas import tpu as pltpu

_K_TOP = 128
_LOCAL_K = 64
_G = 8
_NEG = -3.0e38


def _score_body(qp_ref, w_ref, k_ref, kg_ref, idx_ref, rem_ref,
                *, S, D, local_k, n_top, G):
    n_nl = n_top - local_k
    f32 = jnp.float32
    bf = jnp.bfloat16
    big = jnp.int32(2147483647)

    lane = jax.lax.broadcasted_iota(jnp.int32, (G, S), 1)
    is_local = lane >= (S - local_k)
    row = jax.lax.broadcasted_iota(jnp.int32, (G, S), 0)
    lane64 = jax.lax.broadcasted_iota(jnp.int32, (G, n_nl), 1)
    oh_lane = jax.lax.broadcasted_iota(jnp.int32, (n_nl, S), 1)

    kb = []
    scores = None
    for g in range(G):
        Wb = w_ref[g].astype(bf)
        kbg = k_ref[g].astype(bf)
        kp = jax.lax.dot_general(kbg, Wb, (((1,), (0,)), ((), ())),
                                 preferred_element_type=f32)
        qpg = qp_ref[g].astype(bf)
        sf = jax.lax.dot_general(qpg, kp.astype(bf), (((1,), (1,)), ((), ())),
                                 preferred_element_type=f32)
        scores = sf if g == 0 else jnp.where(row == g, sf, scores)
        kb.append(kbg)
        kg_ref[g, 0:local_k, :] = k_ref[g, S - local_k:S, :]

    scores = scores * (D ** -0.5)
    m8 = jnp.max(scores, axis=1, keepdims=True)
    e8 = jnp.exp(scores - m8)
    se8 = jnp.sum(e8, axis=1, keepdims=True)
    p_loc = jnp.sum(jnp.where(is_local, e8, 0.0), axis=1,
                    keepdims=True) / se8
    ms0 = jnp.where(is_local, _NEG, scores)

    ms = ms0
    mjs8 = jnp.zeros((G, n_nl), f32)
    idxc8 = jnp.zeros((G, n_nl), jnp.int32)
    for j in range(n_nl):
        mj = jnp.max(ms, axis=1, keepdims=True)
        cand = jnp.where(ms == mj, lane, big)
        i8 = jnp.min(cand, axis=1, keepdims=True)
        idxc8 = jnp.where(lane64 == j, i8, idxc8)
        mjs8 = jnp.where(lane64 == j, mj, mjs8)
        ms = jnp.where(lane == i8, _NEG, ms)
    acc8 = jnp.sum(jnp.exp(mjs8 - m8), axis=1, keepdims=True)

    for g in range(G):
        idxcol = idxc8[g:g + 1, :].reshape(n_nl, 1)
        oh = jnp.where(oh_lane == idxcol, 1.0, 0.0).astype(bf)
        rows_k = jax.lax.dot_general(oh, kb[g], (((1,), (0,)), ((), ())),
                                     preferred_element_type=f32)
        kg_ref[g, local_k:n_top, :] = rows_k

    norm = m8 + jnp.log(se8)
    remainder = jnp.log(1.0 - (p_loc + acc8 / se8)) + norm
    rem_ref[0] = jnp.broadcast_to(remainder, (G, 128))
    idx_ref[0] = idxc8


def _value_body(idx_ref, v_ref, vg_ref, mv_ref, *, S, D, local_k, n_top, G):
    n_nl = n_top - local_k
    f32 = jnp.float32
    bf = jnp.bfloat16
    oh_lane = jax.lax.broadcasted_iota(jnp.int32, (n_nl, S), 1)
    for g in range(G):
        v2 = v_ref[g]
        mv_ref[0, g:g + 1, :] = jnp.sum(v2, axis=0, keepdims=True) * (1.0 / S)
        vg_ref[g, 0:local_k, :] = v_ref[g, S - local_k:S, :]
        idxcol = idx_ref[0][g:g + 1, :].reshape(n_nl, 1)
        oh = jnp.where(oh_lane == idxcol, 1.0, 0.0).astype(bf)
        rows_v = jax.lax.dot_general(oh, v2.astype(bf), (((1,), (0,)), ((), ())),
                                     preferred_element_type=f32)
        vg_ref[g, local_k:n_top, :] = rows_v


def kernel(query, key, value, logmask, W):
    B, H, _, D = query.shape
    S = key.shape[2]
    BH = B * H
    R = W.shape[-1]
    G = _G
    NS = BH // G
    qp = jnp.matmul(query, W).reshape(BH, 1, R)
    qp = jnp.broadcast_to(qp, (BH, 8, R))
    k = key.reshape(BH, S, D)
    v = value.reshape(BH, S, D)

    sbody = functools.partial(_score_body, S=S, D=D,
                              local_k=_LOCAL_K, n_top=_K_TOP, G=G)
    kg, idx, rem = pl.pallas_call(
        sbody,
        grid=(NS,),
        in_specs=[
            pl.BlockSpec((G, 8, R), lambda i: (i, 0, 0)),
            pl.BlockSpec((G, D, R), lambda i: (i % (H // G), 0, 0)),
            pl.BlockSpec((G, S, D), lambda i: (i, 0, 0)),
        ],
        out_specs=[
            pl.BlockSpec((G, _K_TOP, D), lambda i: (i, 0, 0)),
            pl.BlockSpec((1, G, _K_TOP - _LOCAL_K), lambda i: (i, 0, 0)),
            pl.BlockSpec((1, G, D), lambda i: (i, 0, 0)),
        ],
        out_shape=[
            jax.ShapeDtypeStruct((BH, _K_TOP, D), jnp.float32),
            jax.ShapeDtypeStruct((NS, G, _K_TOP - _LOCAL_K), jnp.int32),
            jax.ShapeDtypeStruct((NS, G, D), jnp.float32),
        ],
        compiler_params=pltpu.CompilerParams(
            dimension_semantics=("arbitrary",)),
    )(qp, W, k)

    vbody = functools.partial(_value_body, S=S, D=D,
                              local_k=_LOCAL_K, n_top=_K_TOP, G=G)
    vg, mv = pl.pallas_call(
        vbody,
        grid=(NS,),
        in_specs=[
            pl.BlockSpec((1, G, _K_TOP - _LOCAL_K), lambda i: (i, 0, 0)),
            pl.BlockSpec((G, S, D), lambda i: (i, 0, 0)),
        ],
        out_specs=[
            pl.BlockSpec((G, _K_TOP, D), lambda i: (i, 0, 0)),
            pl.BlockSpec((1, G, D), lambda i: (i, 0, 0)),
        ],
        out_shape=[
            jax.ShapeDtypeStruct((BH, _K_TOP, D), jnp.float32),
            jax.ShapeDtypeStruct((NS, G, D), jnp.float32),
        ],
        compiler_params=pltpu.CompilerParams(
            dimension_semantics=("arbitrary",)),
    )(idx, v)

    zeros_row = jnp.zeros((B, H, 1, D), jnp.float32)
    key_out = jnp.concatenate(
        [zeros_row, kg.reshape(B, H, _K_TOP, D)], axis=-2)
    value_out = jnp.concatenate(
        [mv.reshape(B, H, 1, D), vg.reshape(B, H, _K_TOP, D)], axis=-2)
    logmask_out = jnp.concatenate(
        [rem.reshape(B, H, 1, D)[..., :1],
         jnp.zeros((B, H, 1, _K_TOP), logmask.dtype)], axis=-1)
    return (query, key_out, value_out, logmask_out)
